# Initial kernel scaffold; baseline (speedup 1.0000x reference)
import jax
import jax.numpy as jnp
from jax.experimental import pallas as pl


def kernel(x, edge_index, batch, params):
    raise NotImplementedError("write your pallas kernel here")


# speedup vs baseline: 1.1587x; 1.1587x over previous
import functools

import jax
import jax.numpy as jnp
from jax.experimental import pallas as pl

N_NODES = 10000
N_EDGES = 160000
D_IN = 256
HID = 512
LAT = 128
MAXN = 256
NGRAPH = 40

_NODE_BLK = 1000


def _gin_mlp_body(h_ref, agg_ref, w1_ref, b1_ref, w2_ref, b2_ref, o_ref):
    m = h_ref[...] + agg_ref[...]
    t = jax.lax.dot_general(m, w1_ref[...], (((1,), (1,)), ((), ())),
                            preferred_element_type=jnp.float32)
    t = jnp.maximum(t + b1_ref[...], 0.0)
    o = jax.lax.dot_general(t, w2_ref[...], (((1,), (1,)), ((), ())),
                            preferred_element_type=jnp.float32)
    o_ref[...] = jnp.maximum(o + b2_ref[...], 0.0)


@functools.partial(jax.jit, static_argnames=("in_c",))
def _gin_mlp(h, agg, w1, b1, w2, b2, in_c):
    grid = (N_NODES // _NODE_BLK,)
    return pl.pallas_call(
        _gin_mlp_body,
        grid=grid,
        in_specs=[
            pl.BlockSpec((_NODE_BLK, in_c), lambda i: (i, 0)),
            pl.BlockSpec((_NODE_BLK, in_c), lambda i: (i, 0)),
            pl.BlockSpec((HID, in_c), lambda i: (0, 0)),
            pl.BlockSpec((1, HID), lambda i: (0, 0)),
            pl.BlockSpec((HID, HID), lambda i: (0, 0)),
            pl.BlockSpec((1, HID), lambda i: (0, 0)),
        ],
        out_specs=pl.BlockSpec((_NODE_BLK, HID), lambda i: (i, 0)),
        out_shape=jax.ShapeDtypeStruct((N_NODES, HID), jnp.float32),
    )(h, agg, w1, b1.reshape(1, HID), w2, b2.reshape(1, HID))


def _pool_body(h_ref, batch_ref, g_ref):
    @pl.when(pl.program_id(0) == 0)
    def _():
        g_ref[...] = jnp.zeros_like(g_ref)

    seg = batch_ref[pl.program_id(0), :]
    onehot = (seg[None, :] == jax.lax.broadcasted_iota(
        jnp.int32, (NGRAPH, _NODE_BLK), 0)).astype(jnp.float32)
    g_ref[...] += jax.lax.dot_general(
        onehot, h_ref[...], (((1,), (0,)), ((), ())),
        preferred_element_type=jnp.float32)


@jax.jit
def _pool(h, batch2d):
    grid = (N_NODES // _NODE_BLK,)
    return pl.pallas_call(
        _pool_body,
        grid=grid,
        in_specs=[
            pl.BlockSpec((_NODE_BLK, HID), lambda i: (i, 0)),
            pl.BlockSpec((N_NODES // _NODE_BLK, _NODE_BLK), lambda i: (0, 0)),
        ],
        out_specs=pl.BlockSpec((NGRAPH, HID), lambda i: (0, 0)),
        out_shape=jax.ShapeDtypeStruct((NGRAPH, HID), jnp.float32),
    )(h, batch2d)


def _heads_body(g_ref, wmu_ref, bmu_ref, wlv_ref, blv_ref, eps_ref,
                wd1_ref, bd1_ref, mu_ref, lv_ref, hd_ref):
    g = g_ref[...]
    mu = jax.lax.dot_general(g, wmu_ref[...], (((1,), (1,)), ((), ())),
                             preferred_element_type=jnp.float32) + bmu_ref[...]
    lv = jax.lax.dot_general(g, wlv_ref[...], (((1,), (1,)), ((), ())),
                             preferred_element_type=jnp.float32) + blv_ref[...]
    mu_ref[...] = mu
    lv_ref[...] = lv
    z = mu + eps_ref[...] * jnp.exp(0.5 * lv)
    hd = jax.lax.dot_general(z, wd1_ref[...], (((1,), (1,)), ((), ())),
                             preferred_element_type=jnp.float32) + bd1_ref[...]
    hd_ref[...] = jnp.maximum(hd, 0.0)


@jax.jit
def _heads(g, w_mu, b_mu, w_lv, b_lv, eps, w_d1, b_d1):
    return pl.pallas_call(
        _heads_body,
        out_shape=(
            jax.ShapeDtypeStruct((NGRAPH, LAT), jnp.float32),
            jax.ShapeDtypeStruct((NGRAPH, LAT), jnp.float32),
            jax.ShapeDtypeStruct((NGRAPH, HID), jnp.float32),
        ),
    )(g, w_mu, b_mu.reshape(1, LAT), w_lv, b_lv.reshape(1, LAT), eps,
      w_d1, b_d1.reshape(1, HID))


_DEC_BLK = 4096


def _decoder_body(hd_ref, w_ref, b_ref, o_ref):
    o = jax.lax.dot_general(hd_ref[...], w_ref[...], (((1,), (1,)), ((), ())),
                            preferred_element_type=jnp.float32)
    o_ref[...] = o + b_ref[...]


@jax.jit
def _decoder(hd, w_d2, b_d2):
    grid = (MAXN * MAXN // _DEC_BLK,)
    return pl.pallas_call(
        _decoder_body,
        grid=grid,
        in_specs=[
            pl.BlockSpec((NGRAPH, HID), lambda j: (0, 0)),
            pl.BlockSpec((_DEC_BLK, HID), lambda j: (j, 0)),
            pl.BlockSpec((1, _DEC_BLK), lambda j: (0, j)),
        ],
        out_specs=pl.BlockSpec((NGRAPH, _DEC_BLK), lambda j: (0, j)),
        out_shape=jax.ShapeDtypeStruct((NGRAPH, MAXN * MAXN), jnp.float32),
    )(hd, w_d2, b_d2.reshape(1, MAXN * MAXN))


def kernel(x, edge_index, batch, params):
    src, dst = edge_index[0], edge_index[1]
    h = x
    for i, p in enumerate(params['convs']):
        agg = jax.ops.segment_sum(h[src], dst, num_segments=N_NODES)
        in_c = D_IN if i == 0 else HID
        h = _gin_mlp(h, agg, p['w1'], p['b1'], p['w2'], p['b2'], in_c=in_c)

    g = _pool(h, batch.reshape(N_NODES // _NODE_BLK, _NODE_BLK))

    eps = jax.random.normal(jax.random.key(42), (NGRAPH, LAT),
                            dtype=jnp.float32)
    mu, logvar, hd = _heads(g, params['w_mu'], params['b_mu'],
                            params['w_lv'], params['b_lv'], eps,
                            params['w_d1'], params['b_d1'])

    adj = _decoder(hd, params['w_d2'], params['b_d2'])
    adj = adj.reshape(NGRAPH, MAXN, MAXN)
    adj = (adj + jnp.transpose(adj, (0, 2, 1))) / 2.0

    counts = jnp.bincount(batch, length=NGRAPH)
    cum = jnp.concatenate([jnp.zeros((1,), counts.dtype),
                           jnp.cumsum(counts)])[:NGRAPH]
    i0 = batch[src]
    i1 = src - cum[i0]
    i2 = dst - cum[batch[dst]]
    mask = (i1 >= 0) & (i1 < MAXN) & (i2 >= 0) & (i2 < MAXN)
    i1c = jnp.where(mask, i1, 0)
    i2c = jnp.where(mask, i2, 0)
    vals = mask.astype(jnp.float32)
    A = jnp.zeros((NGRAPH, MAXN, MAXN), dtype=jnp.float32).at[
        i0, i1c, i2c].add(vals)

    return (adj, A, mu, logvar)


# speedup vs baseline: 1.1741x; 1.0133x over previous
import functools

import jax
import jax.numpy as jnp
from jax.experimental import pallas as pl

N_NODES = 10000
N_EDGES = 160000
D_IN = 256
HID = 512
LAT = 128
MAXN = 256
NGRAPH = 40

_NODE_BLK = 1000


def _gin_mlp_body(h_ref, agg_ref, w1_ref, b1_ref, w2_ref, b2_ref, o_ref):
    m = h_ref[...] + agg_ref[...]
    t = jax.lax.dot_general(m, w1_ref[...], (((1,), (1,)), ((), ())),
                            preferred_element_type=jnp.float32)
    t = jnp.maximum(t + b1_ref[...], 0.0)
    o = jax.lax.dot_general(t, w2_ref[...], (((1,), (1,)), ((), ())),
                            preferred_element_type=jnp.float32)
    o_ref[...] = jnp.maximum(o + b2_ref[...], 0.0)


@functools.partial(jax.jit, static_argnames=("in_c",))
def _gin_mlp(h, agg, w1, b1, w2, b2, in_c):
    grid = (N_NODES // _NODE_BLK,)
    return pl.pallas_call(
        _gin_mlp_body,
        grid=grid,
        in_specs=[
            pl.BlockSpec((_NODE_BLK, in_c), lambda i: (i, 0)),
            pl.BlockSpec((_NODE_BLK, in_c), lambda i: (i, 0)),
            pl.BlockSpec((HID, in_c), lambda i: (0, 0)),
            pl.BlockSpec((1, HID), lambda i: (0, 0)),
            pl.BlockSpec((HID, HID), lambda i: (0, 0)),
            pl.BlockSpec((1, HID), lambda i: (0, 0)),
        ],
        out_specs=pl.BlockSpec((_NODE_BLK, HID), lambda i: (i, 0)),
        out_shape=jax.ShapeDtypeStruct((N_NODES, HID), jnp.float32),
    )(h, agg, w1, b1.reshape(1, HID), w2, b2.reshape(1, HID))


def _pool_body(h_ref, batch_ref, g_ref):
    @pl.when(pl.program_id(0) == 0)
    def _():
        g_ref[...] = jnp.zeros_like(g_ref)

    seg = batch_ref[pl.program_id(0), :]
    onehot = (seg[None, :] == jax.lax.broadcasted_iota(
        jnp.int32, (NGRAPH, _NODE_BLK), 0)).astype(jnp.float32)
    g_ref[...] += jax.lax.dot_general(
        onehot, h_ref[...], (((1,), (0,)), ((), ())),
        preferred_element_type=jnp.float32)


@jax.jit
def _pool(h, batch2d):
    grid = (N_NODES // _NODE_BLK,)
    return pl.pallas_call(
        _pool_body,
        grid=grid,
        in_specs=[
            pl.BlockSpec((_NODE_BLK, HID), lambda i: (i, 0)),
            pl.BlockSpec((N_NODES // _NODE_BLK, _NODE_BLK), lambda i: (0, 0)),
        ],
        out_specs=pl.BlockSpec((NGRAPH, HID), lambda i: (0, 0)),
        out_shape=jax.ShapeDtypeStruct((NGRAPH, HID), jnp.float32),
    )(h, batch2d)


def _heads_body(g_ref, wmu_ref, bmu_ref, wlv_ref, blv_ref, eps_ref,
                wd1_ref, bd1_ref, mu_ref, lv_ref, hd_ref):
    g = g_ref[...]
    mu = jax.lax.dot_general(g, wmu_ref[...], (((1,), (1,)), ((), ())),
                             preferred_element_type=jnp.float32) + bmu_ref[...]
    lv = jax.lax.dot_general(g, wlv_ref[...], (((1,), (1,)), ((), ())),
                             preferred_element_type=jnp.float32) + blv_ref[...]
    mu_ref[...] = mu
    lv_ref[...] = lv
    z = mu + eps_ref[...] * jnp.exp(0.5 * lv)
    hd = jax.lax.dot_general(z, wd1_ref[...], (((1,), (1,)), ((), ())),
                             preferred_element_type=jnp.float32) + bd1_ref[...]
    hd_ref[...] = jnp.maximum(hd, 0.0)


@jax.jit
def _heads(g, w_mu, b_mu, w_lv, b_lv, eps, w_d1, b_d1):
    return pl.pallas_call(
        _heads_body,
        out_shape=(
            jax.ShapeDtypeStruct((NGRAPH, LAT), jnp.float32),
            jax.ShapeDtypeStruct((NGRAPH, LAT), jnp.float32),
            jax.ShapeDtypeStruct((NGRAPH, HID), jnp.float32),
        ),
    )(g, w_mu, b_mu.reshape(1, LAT), w_lv, b_lv.reshape(1, LAT), eps,
      w_d1, b_d1.reshape(1, HID))


_DEC_BLK = 4096


def _decoder_body(hd_ref, w_ref, b_ref, o_ref):
    o = jax.lax.dot_general(hd_ref[...], w_ref[...], (((1,), (1,)), ((), ())),
                            preferred_element_type=jnp.float32)
    o_ref[...] = o + b_ref[...]


@jax.jit
def _decoder(hd, w_d2, b_d2):
    grid = (MAXN * MAXN // _DEC_BLK,)
    return pl.pallas_call(
        _decoder_body,
        grid=grid,
        in_specs=[
            pl.BlockSpec((NGRAPH, HID), lambda j: (0, 0)),
            pl.BlockSpec((_DEC_BLK, HID), lambda j: (j, 0)),
            pl.BlockSpec((1, _DEC_BLK), lambda j: (0, j)),
        ],
        out_specs=pl.BlockSpec((NGRAPH, _DEC_BLK), lambda j: (0, j)),
        out_shape=jax.ShapeDtypeStruct((NGRAPH, MAXN * MAXN), jnp.float32),
    )(hd, w_d2, b_d2.reshape(1, MAXN * MAXN))


def kernel(x, edge_index, batch, params):
    src, dst = edge_index[0], edge_index[1]
    perm = jnp.argsort(dst)
    dst_s = dst[perm]
    src_s = src[perm]
    h = x
    for i, p in enumerate(params['convs']):
        agg = jax.ops.segment_sum(h[src_s], dst_s, num_segments=N_NODES,
                                  indices_are_sorted=True)
        in_c = D_IN if i == 0 else HID
        h = _gin_mlp(h, agg, p['w1'], p['b1'], p['w2'], p['b2'], in_c=in_c)

    g = _pool(h, batch.reshape(N_NODES // _NODE_BLK, _NODE_BLK))

    eps = jax.random.normal(jax.random.key(42), (NGRAPH, LAT),
                            dtype=jnp.float32)
    mu, logvar, hd = _heads(g, params['w_mu'], params['b_mu'],
                            params['w_lv'], params['b_lv'], eps,
                            params['w_d1'], params['b_d1'])

    adj = _decoder(hd, params['w_d2'], params['b_d2'])
    adj = adj.reshape(NGRAPH, MAXN, MAXN)
    adj = (adj + jnp.transpose(adj, (0, 2, 1))) / 2.0

    counts = jnp.bincount(batch, length=NGRAPH)
    cum = jnp.concatenate([jnp.zeros((1,), counts.dtype),
                           jnp.cumsum(counts)])[:NGRAPH]
    i0 = batch[src]
    i1 = src - cum[i0]
    i2 = dst - cum[batch[dst]]
    mask = (i1 >= 0) & (i1 < MAXN) & (i2 >= 0) & (i2 < MAXN)
    i1c = jnp.where(mask, i1, 0)
    i2c = jnp.where(mask, i2, 0)
    vals = mask.astype(jnp.float32)
    A = jnp.zeros((NGRAPH, MAXN, MAXN), dtype=jnp.float32).at[
        i0, i1c, i2c].add(vals)

    return (adj, A, mu, logvar)
